# Initial kernel scaffold; baseline (speedup 1.0000x reference)
import functools

import jax
import jax.numpy as jnp
from jax import lax
from jax.experimental import pallas as pl
from jax.experimental.pallas import tpu as pltpu
from jax.experimental.pallas import tpu_sc as plsc

BATCH = 16384
HIST = 50
EMBED_DIM = 64

NC = 2
NS = 16
NW = NC * NS

GROUP = 128
TOTAL = BATCH * HIST
N_GROUPS = TOTAL // GROUP
G_PER_W = N_GROUPS // NW


def _make_kernel():
  mesh = plsc.VectorSubcoreMesh(core_axis_name="c", subcore_axis_name="s")

  @functools.partial(
      pl.kernel,
      mesh=mesh,
      out_type=jax.ShapeDtypeStruct((TOTAL, EMBED_DIM), jnp.float32),
      scratch_types=[
          pltpu.VMEM((GROUP,), jnp.int32),
          pltpu.VMEM((GROUP, EMBED_DIM), jnp.float32),
          pltpu.SemaphoreType.DMA,
      ],
  )
  def body(table_hbm, idx_hbm, out_hbm, idx_v, rows_v, sem):
    wid = lax.axis_index("s") * NC + lax.axis_index("c")

    def step(g, carry):
      row = wid * G_PER_W + g
      pltpu.sync_copy(idx_hbm.at[row], idx_v)
      pltpu.async_copy(table_hbm.at[idx_v], rows_v, sem).wait()
      pltpu.sync_copy(rows_v, out_hbm.at[pl.ds(row * GROUP, GROUP)])
      return carry

    lax.fori_loop(0, G_PER_W, step, 0)

  return body


_kernel = _make_kernel()


def kernel(input, table):
  idx = input.astype(jnp.int32).reshape(N_GROUPS, GROUP)
  out = _kernel(table, idx)
  return out.reshape(BATCH, HIST, EMBED_DIM)


# speedup vs baseline: 1.5727x; 1.5727x over previous
import functools

import jax
import jax.numpy as jnp
from jax import lax
from jax.experimental import pallas as pl
from jax.experimental.pallas import tpu as pltpu
from jax.experimental.pallas import tpu_sc as plsc

BATCH = 16384
HIST = 50
EMBED_DIM = 64

NC = 2
NS = 16
NW = NC * NS

GROUP = 128
TOTAL = BATCH * HIST
N_GROUPS = TOTAL // GROUP
G_PER_W = N_GROUPS // NW


def _make_kernel():
  mesh = plsc.VectorSubcoreMesh(core_axis_name="c", subcore_axis_name="s")

  @functools.partial(
      pl.kernel,
      mesh=mesh,
      out_type=jax.ShapeDtypeStruct((TOTAL, EMBED_DIM), jnp.float32),
      compiler_params=pltpu.CompilerParams(use_tc_tiling_on_sc=False),
      scratch_types=[
          pltpu.VMEM((GROUP,), jnp.int32),
          pltpu.VMEM((GROUP, EMBED_DIM), jnp.float32),
          pltpu.SemaphoreType.DMA,
      ],
  )
  def body(table_hbm, idx_hbm, out_hbm, idx_v, rows_v, sem):
    wid = lax.axis_index("s") * NC + lax.axis_index("c")

    def step(g, carry):
      row = wid * G_PER_W + g
      pltpu.sync_copy(idx_hbm.at[row], idx_v)
      pltpu.async_copy(table_hbm.at[idx_v], rows_v, sem).wait()
      pltpu.sync_copy(rows_v, out_hbm.at[pl.ds(row * GROUP, GROUP)])
      return carry

    lax.fori_loop(0, G_PER_W, step, 0)

  return body


_kernel = _make_kernel()


def kernel(input, table):
  idx = input.astype(jnp.int32).reshape(N_GROUPS, GROUP)
  out = _kernel(table, idx)
  return out.reshape(BATCH, HIST, EMBED_DIM)


# speedup vs baseline: 1.8788x; 1.1947x over previous
import functools

import jax
import jax.numpy as jnp
from jax import lax
from jax.experimental import pallas as pl
from jax.experimental.pallas import tpu as pltpu
from jax.experimental.pallas import tpu_sc as plsc

BATCH = 16384
HIST = 50
EMBED_DIM = 64

NC = 2
NS = 16
NW = NC * NS

GROUP = 128
TOTAL = BATCH * HIST
N_GROUPS = TOTAL // GROUP
G_PER_W = N_GROUPS // NW
NBUF = 4


def _make_kernel():
  mesh = plsc.VectorSubcoreMesh(core_axis_name="c", subcore_axis_name="s")

  @functools.partial(
      pl.kernel,
      mesh=mesh,
      out_type=jax.ShapeDtypeStruct((TOTAL, EMBED_DIM), jnp.float32),
      compiler_params=pltpu.CompilerParams(use_tc_tiling_on_sc=False),
      scratch_types=[
          pltpu.VMEM((G_PER_W, GROUP), jnp.int32),
          [pltpu.VMEM((GROUP, EMBED_DIM), jnp.float32) for _ in range(NBUF)],
          [pltpu.SemaphoreType.DMA for _ in range(NBUF)],
          [pltpu.SemaphoreType.DMA for _ in range(NBUF)],
      ],
  )
  def body(table_hbm, idx_hbm, out_hbm, idx_v, rows, gsem, osem):
    wid = lax.axis_index("s") * NC + lax.axis_index("c")
    base = wid * G_PER_W

    pltpu.sync_copy(idx_hbm.at[pl.ds(base, G_PER_W)], idx_v)

    def gather(g, b):
      pltpu.async_copy(table_hbm.at[idx_v.at[g]], rows[b], gsem[b])

    def put(g, b):
      pltpu.async_copy(
          rows[b], out_hbm.at[pl.ds((base + g) * GROUP, GROUP)], osem[b])

    for b in range(NBUF):
      gather(b, b)

    def outer(j, carry):
      for b in range(NBUF):
        g = j * NBUF + b
        pltpu.make_async_copy(
            table_hbm.at[idx_v.at[g]], rows[b], gsem[b]).wait()
        put(g, b)
        pltpu.make_async_copy(
            rows[b], out_hbm.at[pl.ds((base + g) * GROUP, GROUP)],
            osem[b]).wait()
        gather(g + NBUF, b)
      return carry

    lax.fori_loop(0, G_PER_W // NBUF - 1, outer, 0)

    last = G_PER_W - NBUF
    for b in range(NBUF):
      g = last + b
      pltpu.make_async_copy(
          table_hbm.at[idx_v.at[g]], rows[b], gsem[b]).wait()
      put(g, b)
    for b in range(NBUF):
      g = last + b
      pltpu.make_async_copy(
          rows[b], out_hbm.at[pl.ds((base + g) * GROUP, GROUP)],
          osem[b]).wait()

  return body


_kernel = _make_kernel()


def kernel(input, table):
  idx = input.astype(jnp.int32).reshape(N_GROUPS, GROUP)
  out = _kernel(table, idx)
  return out.reshape(BATCH, HIST, EMBED_DIM)
